# baseline (speedup 1.0000x reference)
import functools

import jax
import jax.numpy as jnp
from jax import lax
from jax.experimental import pallas as pl
from jax.experimental.pallas import tpu as pltpu, tpu_sc as plsc

EMBED_DIM = 64
NC, NS = 2, 16
NW = NC * NS
BLK = 128


def _gather_body(n_chunks, table_hbm, ids_hbm, out_hbm,
                 idx0, idx1, rows0, rows1,
                 isem0, isem1, gsem0, gsem1, osem0, osem1):
    wid = lax.axis_index("s") * NC + lax.axis_index("c")
    base = wid * (n_chunks * BLK)
    idxv = (idx0, idx1)
    rows = (rows0, rows1)
    isem = (isem0, isem1)
    gsem = (gsem0, gsem1)
    osem = (osem0, osem1)

    def _idx_src(u):
        return ids_hbm.at[pl.ds(base + u * BLK, BLK)]

    def _out_dst(u):
        return out_hbm.at[pl.ds(base + u * BLK, BLK), :]

    def _fire_idx(u, b):
        pltpu.async_copy(_idx_src(u), idxv[b], isem[b])

    def _wait_idx(u, b):
        pltpu.make_async_copy(_idx_src(u), idxv[b], isem[b]).wait()

    def _fire_gather(b):
        pltpu.async_copy(table_hbm.at[idxv[b]], rows[b], gsem[b])

    def _wait_gather(b):
        pltpu.make_async_copy(table_hbm.at[idxv[b]], rows[b], gsem[b]).wait()

    def _fire_out(u, b):
        pltpu.async_copy(rows[b], _out_dst(u), osem[b])

    def _wait_out(u, b):
        pltpu.make_async_copy(rows[b], _out_dst(u), osem[b]).wait()

    _fire_idx(0, 0)
    _fire_idx(1, 1)
    _wait_idx(0, 0)
    _fire_gather(0)

    n2 = n_chunks // 2

    @pl.loop(0, n2)
    def _pair(p):
        for b in (0, 1):
            u = 2 * p + b
            nb = 1 - b
            if b == 0:
                _wait_idx(u + 1, nb)

                @pl.when(p >= 1)
                def _():
                    _wait_out(u - 1, nb)

                _fire_gather(nb)
            else:
                @pl.when(p < n2 - 1)
                def _():
                    _wait_idx(u + 1, nb)
                    _wait_out(u - 1, nb)
                    _fire_gather(nb)

            _wait_gather(b)

            @pl.when(p < n2 - 1)
            def _():
                _fire_idx(u + 2, b)

            _fire_out(u, b)

    _wait_out(n_chunks - 2, 0)
    _wait_out(n_chunks - 1, 1)


@jax.jit
def kernel(input_ids, table):
    b, s = input_ids.shape
    ids_flat = jnp.reshape(input_ids, (b * s,)).astype(jnp.int32)
    n_chunks = (b * s) // (NW * BLK)
    call = pl.kernel(
        functools.partial(_gather_body, n_chunks),
        out_type=jax.ShapeDtypeStruct((b * s, EMBED_DIM), jnp.float32),
        mesh=plsc.VectorSubcoreMesh(
            core_axis_name="c", subcore_axis_name="s",
            num_cores=NC, num_subcores=NS,
        ),
        scratch_types=[
            pltpu.VMEM((BLK,), jnp.int32),
            pltpu.VMEM((BLK,), jnp.int32),
            pltpu.VMEM((BLK, EMBED_DIM), jnp.float32),
            pltpu.VMEM((BLK, EMBED_DIM), jnp.float32),
            pltpu.SemaphoreType.DMA,
            pltpu.SemaphoreType.DMA,
            pltpu.SemaphoreType.DMA,
            pltpu.SemaphoreType.DMA,
            pltpu.SemaphoreType.DMA,
            pltpu.SemaphoreType.DMA,
        ],
        compiler_params=pltpu.CompilerParams(use_tc_tiling_on_sc=False),
    )
    out2 = call(table, ids_flat)
    return jnp.reshape(out2, (b, s, EMBED_DIM))


# speedup vs baseline: 1.0317x; 1.0317x over previous
import functools

import jax
import jax.numpy as jnp
from jax import lax
from jax.experimental import pallas as pl
from jax.experimental.pallas import tpu as pltpu, tpu_sc as plsc

EMBED_DIM = 64
NC, NS = 2, 16
NW = NC * NS
BLK = 512


def _gather_body(n_chunks, table_hbm, ids_hbm, out_hbm,
                 idx0, idx1, rows0, rows1,
                 isem0, isem1, gsem0, gsem1, osem0, osem1):
    wid = lax.axis_index("s") * NC + lax.axis_index("c")
    base = wid * (n_chunks * BLK)
    idxv = (idx0, idx1)
    rows = (rows0, rows1)
    isem = (isem0, isem1)
    gsem = (gsem0, gsem1)
    osem = (osem0, osem1)

    def _idx_src(u):
        return ids_hbm.at[pl.ds(base + u * BLK, BLK)]

    def _out_dst(u):
        return out_hbm.at[pl.ds(base + u * BLK, BLK), :]

    def _fire_idx(u, b):
        pltpu.async_copy(_idx_src(u), idxv[b], isem[b])

    def _wait_idx(u, b):
        pltpu.make_async_copy(_idx_src(u), idxv[b], isem[b]).wait()

    def _fire_gather(b):
        pltpu.async_copy(table_hbm.at[idxv[b]], rows[b], gsem[b])

    def _wait_gather(b):
        pltpu.make_async_copy(table_hbm.at[idxv[b]], rows[b], gsem[b]).wait()

    def _fire_out(u, b):
        pltpu.async_copy(rows[b], _out_dst(u), osem[b])

    def _wait_out(u, b):
        pltpu.make_async_copy(rows[b], _out_dst(u), osem[b]).wait()

    _fire_idx(0, 0)
    _fire_idx(1, 1)
    _wait_idx(0, 0)
    _fire_gather(0)

    n2 = n_chunks // 2

    @pl.loop(0, n2)
    def _pair(p):
        for b in (0, 1):
            u = 2 * p + b
            nb = 1 - b
            if b == 0:
                _wait_idx(u + 1, nb)

                @pl.when(p >= 1)
                def _():
                    _wait_out(u - 1, nb)

                _fire_gather(nb)
            else:
                @pl.when(p < n2 - 1)
                def _():
                    _wait_idx(u + 1, nb)
                    _wait_out(u - 1, nb)
                    _fire_gather(nb)

            _wait_gather(b)

            @pl.when(p < n2 - 1)
            def _():
                _fire_idx(u + 2, b)

            _fire_out(u, b)

    _wait_out(n_chunks - 2, 0)
    _wait_out(n_chunks - 1, 1)


@jax.jit
def kernel(input_ids, table):
    b, s = input_ids.shape
    ids_flat = jnp.reshape(input_ids, (b * s,)).astype(jnp.int32)
    n_chunks = (b * s) // (NW * BLK)
    call = pl.kernel(
        functools.partial(_gather_body, n_chunks),
        out_type=jax.ShapeDtypeStruct((b * s, EMBED_DIM), jnp.float32),
        mesh=plsc.VectorSubcoreMesh(
            core_axis_name="c", subcore_axis_name="s",
            num_cores=NC, num_subcores=NS,
        ),
        scratch_types=[
            pltpu.VMEM((BLK,), jnp.int32),
            pltpu.VMEM((BLK,), jnp.int32),
            pltpu.VMEM((BLK, EMBED_DIM), jnp.float32),
            pltpu.VMEM((BLK, EMBED_DIM), jnp.float32),
            pltpu.SemaphoreType.DMA,
            pltpu.SemaphoreType.DMA,
            pltpu.SemaphoreType.DMA,
            pltpu.SemaphoreType.DMA,
            pltpu.SemaphoreType.DMA,
            pltpu.SemaphoreType.DMA,
        ],
        compiler_params=pltpu.CompilerParams(use_tc_tiling_on_sc=False),
    )
    out2 = call(table, ids_flat)
    return jnp.reshape(out2, (b, s, EMBED_DIM))
